# baseline (speedup 1.0000x reference)
import functools
import math

import jax
import jax.numpy as jnp
from jax.experimental import pallas as pl
from jax.experimental.pallas import tpu as pltpu
from jax.experimental.pallas import tpu_sc as plsc

J = 17
T = 34
H = 512
C = 1024
D = 512
BS = 256
BB = 8
G = BS // BB
R = BB * J
RT = BB * T
NZ = BS * T
DH = 32
EPS = 1e-5

_GW = 16


def _ln(x, g, b):
    m = jnp.mean(x, -1, keepdims=True)
    v = jnp.mean((x - m) ** 2, -1, keepdims=True)
    return (x - m) / jnp.sqrt(v + EPS) * g + b


def _gelu(x):
    return x * 0.5 * (1.0 + jax.lax.erf(x * (1.0 / math.sqrt(2.0))))


def _enc_kernel(coords, w, inv, sw, sb, *rest):
    blocks = [rest[12 * k:12 * (k + 1)] for k in range(4)]
    lng, lnb, mt, mtb, few, feb, cbt = rest[48:55]
    z_ref, q_ref, cbsq_ref = rest[55:58]

    i = pl.program_id(0)

    @pl.when(i == 0)
    def _():
        cbsq_ref[...] = jnp.sum(cbt[...] * cbt[...], axis=0, keepdims=True)

    wv = w[...]
    feat = jnp.dot(coords[...], sw[...]) + sb[...]
    feat = feat * wv + inv[...] * (1.0 - wv)

    for (l1g, l1b, m1, t1b, m2, t2b, l2g, l2b, c1w, c1b, c2w, c2b) in blocks:
        y = _ln(feat, l1g[...], l1b[...])
        h = _gelu(jnp.dot(m1[...], y) + t1b[...])
        y = jnp.dot(m2[...], h) + t2b[...]
        zin = _ln(feat + y, l2g[...], l2b[...])
        hh = _gelu(jnp.dot(zin, c1w[...]) + c1b[...])
        zz = jnp.dot(hh, c2w[...]) + c2b[...]
        feat = feat + y + zz

    feat = _ln(feat, lng[...], lnb[...])
    tk = jnp.dot(mt[...], feat) + mtb[...]
    z = jnp.dot(tk, few[...]) + feb[...]

    zsq = jnp.sum(z * z, axis=1, keepdims=True)
    d2 = zsq - 2.0 * jnp.dot(z, cbt[...]) + cbsq_ref[...]
    dmin = jnp.min(d2, axis=1, keepdims=True)
    lanes = jax.lax.broadcasted_iota(jnp.int32, d2.shape, 1)
    q = jnp.min(jnp.where(d2 == dmin, lanes, C), axis=1, keepdims=True)

    z_ref[...] = z
    q_ref[...] = q


def _dec_kernel(z, zq, md, mdb, dsw, dsb,
                l1g, l1b, dm1, dt1b, dm2, dt2b, l2g, l2b,
                dc1w, dc1b, dc2w, dc2b,
                lng, lnb, rw, rb, rec_ref, lsum_ref):
    i = pl.program_id(0)
    zv = z[...]
    zqv = zq[...]

    @pl.when(i == 0)
    def _():
        lsum_ref[...] = jnp.zeros_like(lsum_ref)

    diff = zv - zqv
    lsum_ref[...] += jnp.sum(diff * diff, axis=(0, 1), keepdims=True)

    ste = zv + (zqv - zv)
    part = jnp.dot(md[...], ste) + mdb[...]
    dec = jnp.dot(part, dsw[...]) + dsb[...]

    y = _ln(dec, l1g[...], l1b[...])
    h = _gelu(jnp.dot(dm1[...], y) + dt1b[...])
    y = jnp.dot(dm2[...], h) + dt2b[...]
    zin = _ln(dec + y, l2g[...], l2b[...])
    hh = _gelu(jnp.dot(zin, dc1w[...]) + dc1b[...])
    zz = jnp.dot(hh, dc2w[...]) + dc2b[...]
    dec = dec + y + zz

    dec = _ln(dec, lng[...], lnb[...])
    rec_ref[...] = jnp.dot(dec, rw[...]) + rb[...]


def _const2(shape):
    return pl.BlockSpec(shape, lambda i: (0, 0))


_NW = 32
_BPW = NZ // _NW
_CH = 16
_K = 8


def _sc_gather(cb, q, n=NZ):
    mesh = plsc.VectorSubcoreMesh(core_axis_name="c", subcore_axis_name="s")
    grp = _K * _CH
    bpw = n // _NW
    assert bpw * _NW == n and bpw % 8 == 0

    groups = []
    off = 0
    while off < bpw:
        g = []
        while off < bpw and len(g) < _K:
            sz = min(_CH, bpw - off)
            g.append((off, sz))
            off += sz
        groups.append(g)

    @functools.partial(
        pl.kernel,
        out_type=jax.ShapeDtypeStruct((n, D), cb.dtype),
        mesh=mesh,
        scratch_types=[
            pltpu.VMEM((bpw,), jnp.int32),
            pltpu.VMEM((grp, D), jnp.float32),
            pltpu.SemaphoreType.DMA,
        ],
    )
    def kern(cb_hbm, q_hbm, o_hbm, idx_v, rows_v, sem):
        wid = jax.lax.axis_index("s") * 2 + jax.lax.axis_index("c")
        base = wid * bpw
        pltpu.sync_copy(q_hbm.at[pl.ds(base, bpw)], idx_v)
        for g in groups:
            cps = []
            g0 = g[0][0]
            for off, sz in g:
                cps.append(pltpu.async_copy(
                    cb_hbm.at[idx_v.at[pl.ds(off, sz)]],
                    rows_v.at[pl.ds(off - g0, sz)], sem))
            for cp in cps:
                cp.wait()
            gn = g[-1][0] + g[-1][1] - g0
            pltpu.sync_copy(rows_v.at[pl.ds(0, gn)],
                            o_hbm.at[pl.ds(base + g0, gn)])

    return kern(cb, q)


def _row(b):
    return b.reshape(1, -1)


def _bd(wt, bb=BB):
    return jnp.kron(jnp.eye(bb, dtype=wt.dtype), wt.T)


def _colb(b, bb=BB):
    return jnp.tile(b, bb).reshape(-1, 1)


def kernel(joints, joints_feature, cls_logits, params):
    del joints_feature, cls_logits
    coords = joints[:, :, :2].reshape(BS * J, 2)
    w2d = (joints[:, :, 2] != 0).astype(jnp.float32).reshape(BS * J, 1)

    enc_args = [coords, w2d,
                params["invisible_token"].reshape(1, H),
                params["start_embed"]["w"], _row(params["start_embed"]["b"])]
    enc_specs = [
        pl.BlockSpec((R, 2), lambda i: (i, 0)),
        pl.BlockSpec((R, 1), lambda i: (i, 0)),
        _const2((1, H)), _const2((2, H)), _const2((1, H)),
    ]
    for p in params["encoder"]:
        enc_args += [
            _row(p["ln1_g"]), _row(p["ln1_b"]),
            _bd(p["tok1"]["w"]), _colb(p["tok1"]["b"]),
            _bd(p["tok2"]["w"]), _colb(p["tok2"]["b"]),
            _row(p["ln2_g"]), _row(p["ln2_b"]),
            p["ch1"]["w"], _row(p["ch1"]["b"]),
            p["ch2"]["w"], _row(p["ch2"]["b"]),
        ]
        enc_specs += [
            _const2((1, H)), _const2((1, H)),
            _const2((BB * 64, R)), _const2((BB * 64, 1)),
            _const2((R, BB * 64)), _const2((R, 1)),
            _const2((1, H)), _const2((1, H)),
            _const2((H, H)), _const2((1, H)),
            _const2((H, H)), _const2((1, H)),
        ]
    enc_args += [
        _row(params["enc_ln_g"]), _row(params["enc_ln_b"]),
        _bd(params["token_mlp"]["w"]), _colb(params["token_mlp"]["b"]),
        params["feature_embed"]["w"], _row(params["feature_embed"]["b"]),
        params["codebook"].T,
    ]
    enc_specs += [
        _const2((1, H)), _const2((1, H)),
        _const2((RT, R)), _const2((RT, 1)),
        _const2((H, D)), _const2((1, D)),
        _const2((D, C)),
    ]

    z2d, q2d = pl.pallas_call(
        _enc_kernel,
        grid=(G,),
        in_specs=enc_specs,
        out_specs=[
            pl.BlockSpec((RT, D), lambda i: (i, 0)),
            pl.BlockSpec((RT, 1), lambda i: (i, 0)),
        ],
        out_shape=[
            jax.ShapeDtypeStruct((NZ, D), jnp.float32),
            jax.ShapeDtypeStruct((NZ, 1), jnp.int32),
        ],
        scratch_shapes=[pltpu.VMEM((1, C), jnp.float32)],
    )(*enc_args)

    q = q2d.reshape(NZ)
    z_q_a = _sc_gather(params["codebook"], q[:NZ // 2], NZ // 2)
    z_q_b = jnp.take(params["codebook"], q[NZ // 2:], axis=0)
    z_q = jnp.concatenate([z_q_a, z_q_b], axis=0)

    dp = params["decoder"][0]
    dec_args = [
        z2d, z_q,
        _bd(params["decoder_token_mlp"]["w"]),
        _colb(params["decoder_token_mlp"]["b"]),
        params["decoder_start"]["w"], _row(params["decoder_start"]["b"]),
        _row(dp["ln1_g"]), _row(dp["ln1_b"]),
        _bd(dp["tok1"]["w"]), _colb(dp["tok1"]["b"]),
        _bd(dp["tok2"]["w"]), _colb(dp["tok2"]["b"]),
        _row(dp["ln2_g"]), _row(dp["ln2_b"]),
        dp["ch1"]["w"], _row(dp["ch1"]["b"]),
        dp["ch2"]["w"], _row(dp["ch2"]["b"]),
        _row(params["dec_ln_g"]), _row(params["dec_ln_b"]),
        params["recover_embed"]["w"], _row(params["recover_embed"]["b"]),
    ]
    dec_specs = [
        pl.BlockSpec((RT, D), lambda i: (i, 0)),
        pl.BlockSpec((RT, D), lambda i: (i, 0)),
        _const2((R, RT)), _const2((R, 1)),
        _const2((D, DH)), _const2((1, DH)),
        _const2((1, DH)), _const2((1, DH)),
        _const2((BB * 64, R)), _const2((BB * 64, 1)),
        _const2((R, BB * 64)), _const2((R, 1)),
        _const2((1, DH)), _const2((1, DH)),
        _const2((DH, 64)), _const2((1, 64)),
        _const2((64, DH)), _const2((1, DH)),
        _const2((1, DH)), _const2((1, DH)),
        _const2((DH, 2)), _const2((1, 2)),
    ]

    rec2d, lsum = pl.pallas_call(
        _dec_kernel,
        grid=(G,),
        in_specs=dec_specs,
        out_specs=[
            pl.BlockSpec((R, 2), lambda i: (i, 0)),
            pl.BlockSpec((1, 1), lambda i: (0, 0)),
        ],
        out_shape=[
            jax.ShapeDtypeStruct((BS * J, 2), jnp.float32),
            jax.ShapeDtypeStruct((1, 1), jnp.float32),
        ],
    )(*dec_args)

    rec = rec2d.reshape(BS, J, 2)
    e_latent_loss = lsum[0, 0] / (NZ * D)
    return rec, q, e_latent_loss


# speedup vs baseline: 1.0441x; 1.0441x over previous
import functools
import math

import jax
import jax.numpy as jnp
from jax.experimental import pallas as pl
from jax.experimental.pallas import tpu as pltpu
from jax.experimental.pallas import tpu_sc as plsc

J = 17
T = 34
H = 512
C = 1024
D = 512
BS = 256
BB = 8
G = BS // BB
R = BB * J
RT = BB * T
NZ = BS * T
DH = 32
EPS = 1e-5

_GW = 16


def _ln(x, g, b):
    m = jnp.mean(x, -1, keepdims=True)
    v = jnp.mean((x - m) ** 2, -1, keepdims=True)
    return (x - m) / jnp.sqrt(v + EPS) * g + b


def _gelu(x):
    return x * 0.5 * (1.0 + jax.lax.erf(x * (1.0 / math.sqrt(2.0))))


def _enc_kernel(coords, w, inv, sw, sb, *rest):
    blocks = [rest[12 * k:12 * (k + 1)] for k in range(4)]
    lng, lnb, mt, mtb, few, feb, cbt = rest[48:55]
    z_ref, q_ref, cbsq_ref = rest[55:58]

    i = pl.program_id(0)

    @pl.when(i == 0)
    def _():
        cbsq_ref[...] = jnp.sum(cbt[...] * cbt[...], axis=0, keepdims=True)

    wv = w[...]
    feat = jnp.dot(coords[...], sw[...]) + sb[...]
    feat = feat * wv + inv[...] * (1.0 - wv)

    for (l1g, l1b, m1, t1b, m2, t2b, l2g, l2b, c1w, c1b, c2w, c2b) in blocks:
        y = _ln(feat, l1g[...], l1b[...])
        h = _gelu(jnp.dot(m1[...], y) + t1b[...])
        y = jnp.dot(m2[...], h) + t2b[...]
        zin = _ln(feat + y, l2g[...], l2b[...])
        hh = _gelu(jnp.dot(zin, c1w[...]) + c1b[...])
        zz = jnp.dot(hh, c2w[...]) + c2b[...]
        feat = feat + y + zz

    feat = _ln(feat, lng[...], lnb[...])
    tk = jnp.dot(mt[...], feat) + mtb[...]
    z = jnp.dot(tk, few[...]) + feb[...]

    zsq = jnp.sum(z * z, axis=1, keepdims=True)
    d2 = zsq - 2.0 * jnp.dot(z, cbt[...]) + cbsq_ref[...]
    dmin = jnp.min(d2, axis=1, keepdims=True)
    lanes = jax.lax.broadcasted_iota(jnp.int32, d2.shape, 1)
    q = jnp.min(jnp.where(d2 == dmin, lanes, C), axis=1, keepdims=True)

    z_ref[...] = z
    q_ref[...] = q


def _dec_kernel(z, zq, md, mdb, dsw, dsb,
                l1g, l1b, dm1, dt1b, dm2, dt2b, l2g, l2b,
                dc1w, dc1b, dc2w, dc2b,
                lng, lnb, rw, rb, rec_ref, lsum_ref):
    i = pl.program_id(0)
    zv = z[...]
    zqv = zq[...]

    @pl.when(i == 0)
    def _():
        lsum_ref[...] = jnp.zeros_like(lsum_ref)

    diff = zv - zqv
    lsum_ref[...] += jnp.sum(diff * diff, axis=(0, 1), keepdims=True)

    ste = zv + (zqv - zv)
    part = jnp.dot(md[...], ste) + mdb[...]
    dec = jnp.dot(part, dsw[...]) + dsb[...]

    y = _ln(dec, l1g[...], l1b[...])
    h = _gelu(jnp.dot(dm1[...], y) + dt1b[...])
    y = jnp.dot(dm2[...], h) + dt2b[...]
    zin = _ln(dec + y, l2g[...], l2b[...])
    hh = _gelu(jnp.dot(zin, dc1w[...]) + dc1b[...])
    zz = jnp.dot(hh, dc2w[...]) + dc2b[...]
    dec = dec + y + zz

    dec = _ln(dec, lng[...], lnb[...])
    rec_ref[...] = jnp.dot(dec, rw[...]) + rb[...]


def _const2(shape):
    return pl.BlockSpec(shape, lambda i: (0, 0))


_NW = 32
_BPW = NZ // _NW
_CH = 16
_K = 8


def _sc_gather(cb, q, n=NZ):
    mesh = plsc.VectorSubcoreMesh(core_axis_name="c", subcore_axis_name="s")
    grp = _K * _CH
    bpw = n // _NW
    assert bpw * _NW == n and bpw % 8 == 0

    groups = []
    off = 0
    while off < bpw:
        g = []
        while off < bpw and len(g) < _K:
            sz = min(_CH, bpw - off)
            g.append((off, sz))
            off += sz
        groups.append(g)

    @functools.partial(
        pl.kernel,
        out_type=jax.ShapeDtypeStruct((n, D), cb.dtype),
        mesh=mesh,
        scratch_types=[
            pltpu.VMEM((bpw,), jnp.int32),
            pltpu.VMEM((grp, D), jnp.float32),
            pltpu.SemaphoreType.DMA,
        ],
    )
    def kern(cb_hbm, q_hbm, o_hbm, idx_v, rows_v, sem):
        wid = jax.lax.axis_index("s") * 2 + jax.lax.axis_index("c")
        base = wid * bpw
        pltpu.sync_copy(q_hbm.at[pl.ds(base, bpw)], idx_v)
        for g in groups:
            cps = []
            g0 = g[0][0]
            for off, sz in g:
                cps.append(pltpu.async_copy(
                    cb_hbm.at[idx_v.at[pl.ds(off, sz)]],
                    rows_v.at[pl.ds(off - g0, sz)], sem))
            for cp in cps:
                cp.wait()
            gn = g[-1][0] + g[-1][1] - g0
            pltpu.sync_copy(rows_v.at[pl.ds(0, gn)],
                            o_hbm.at[pl.ds(base + g0, gn)])

    return kern(cb, q)


def _row(b):
    return b.reshape(1, -1)


def _bd(wt, bb=BB):
    return jnp.kron(jnp.eye(bb, dtype=wt.dtype), wt.T)


def _colb(b, bb=BB):
    return jnp.tile(b, bb).reshape(-1, 1)


def kernel(joints, joints_feature, cls_logits, params):
    del joints_feature, cls_logits
    coords = joints[:, :, :2].reshape(BS * J, 2)
    w2d = (joints[:, :, 2] != 0).astype(jnp.float32).reshape(BS * J, 1)

    enc_args = [coords, w2d,
                params["invisible_token"].reshape(1, H),
                params["start_embed"]["w"], _row(params["start_embed"]["b"])]
    enc_specs = [
        pl.BlockSpec((R, 2), lambda i: (i, 0)),
        pl.BlockSpec((R, 1), lambda i: (i, 0)),
        _const2((1, H)), _const2((2, H)), _const2((1, H)),
    ]
    for p in params["encoder"]:
        enc_args += [
            _row(p["ln1_g"]), _row(p["ln1_b"]),
            _bd(p["tok1"]["w"]), _colb(p["tok1"]["b"]),
            _bd(p["tok2"]["w"]), _colb(p["tok2"]["b"]),
            _row(p["ln2_g"]), _row(p["ln2_b"]),
            p["ch1"]["w"], _row(p["ch1"]["b"]),
            p["ch2"]["w"], _row(p["ch2"]["b"]),
        ]
        enc_specs += [
            _const2((1, H)), _const2((1, H)),
            _const2((BB * 64, R)), _const2((BB * 64, 1)),
            _const2((R, BB * 64)), _const2((R, 1)),
            _const2((1, H)), _const2((1, H)),
            _const2((H, H)), _const2((1, H)),
            _const2((H, H)), _const2((1, H)),
        ]
    enc_args += [
        _row(params["enc_ln_g"]), _row(params["enc_ln_b"]),
        _bd(params["token_mlp"]["w"]), _colb(params["token_mlp"]["b"]),
        params["feature_embed"]["w"], _row(params["feature_embed"]["b"]),
        params["codebook"].T,
    ]
    enc_specs += [
        _const2((1, H)), _const2((1, H)),
        _const2((RT, R)), _const2((RT, 1)),
        _const2((H, D)), _const2((1, D)),
        _const2((D, C)),
    ]

    NH = NZ // 2
    RH = BS * J // 2
    GH = G // 2

    def _enc_half(coords_h, w_h):
        return pl.pallas_call(
            _enc_kernel,
            grid=(GH,),
            in_specs=enc_specs,
            out_specs=[
                pl.BlockSpec((RT, D), lambda i: (i, 0)),
                pl.BlockSpec((RT, 1), lambda i: (i, 0)),
            ],
            out_shape=[
                jax.ShapeDtypeStruct((NH, D), jnp.float32),
                jax.ShapeDtypeStruct((NH, 1), jnp.int32),
            ],
            scratch_shapes=[pltpu.VMEM((1, C), jnp.float32)],
        )(coords_h, w_h, *enc_args[2:])

    z_h, q_h = [], []
    for h in range(2):
        zh, qh = _enc_half(coords[h * RH:(h + 1) * RH],
                           w2d[h * RH:(h + 1) * RH])
        z_h.append(zh)
        q_h.append(qh.reshape(NH))

    zq_h = [_sc_gather(params["codebook"], q_h[h], NH) for h in range(2)]

    dp = params["decoder"][0]
    dec_args = [
        _bd(params["decoder_token_mlp"]["w"]),
        _colb(params["decoder_token_mlp"]["b"]),
        params["decoder_start"]["w"], _row(params["decoder_start"]["b"]),
        _row(dp["ln1_g"]), _row(dp["ln1_b"]),
        _bd(dp["tok1"]["w"]), _colb(dp["tok1"]["b"]),
        _bd(dp["tok2"]["w"]), _colb(dp["tok2"]["b"]),
        _row(dp["ln2_g"]), _row(dp["ln2_b"]),
        dp["ch1"]["w"], _row(dp["ch1"]["b"]),
        dp["ch2"]["w"], _row(dp["ch2"]["b"]),
        _row(params["dec_ln_g"]), _row(params["dec_ln_b"]),
        params["recover_embed"]["w"], _row(params["recover_embed"]["b"]),
    ]
    dec_specs = [
        pl.BlockSpec((RT, D), lambda i: (i, 0)),
        pl.BlockSpec((RT, D), lambda i: (i, 0)),
        _const2((R, RT)), _const2((R, 1)),
        _const2((D, DH)), _const2((1, DH)),
        _const2((1, DH)), _const2((1, DH)),
        _const2((BB * 64, R)), _const2((BB * 64, 1)),
        _const2((R, BB * 64)), _const2((R, 1)),
        _const2((1, DH)), _const2((1, DH)),
        _const2((DH, 64)), _const2((1, 64)),
        _const2((64, DH)), _const2((1, DH)),
        _const2((1, DH)), _const2((1, DH)),
        _const2((DH, 2)), _const2((1, 2)),
    ]

    def _dec_half(zh, zqh):
        return pl.pallas_call(
            _dec_kernel,
            grid=(GH,),
            in_specs=dec_specs,
            out_specs=[
                pl.BlockSpec((R, 2), lambda i: (i, 0)),
                pl.BlockSpec((1, 1), lambda i: (0, 0)),
            ],
            out_shape=[
                jax.ShapeDtypeStruct((RH, 2), jnp.float32),
                jax.ShapeDtypeStruct((1, 1), jnp.float32),
            ],
        )(zh, zqh, *dec_args)

    rec_h, lsum_h = [], []
    for h in range(2):
        rh, lh = _dec_half(z_h[h], zq_h[h])
        rec_h.append(rh)
        lsum_h.append(lh)

    rec = jnp.concatenate(rec_h, axis=0).reshape(BS, J, 2)
    q = jnp.concatenate(q_h, axis=0)
    e_latent_loss = (lsum_h[0][0, 0] + lsum_h[1][0, 0]) / (NZ * D)
    return rec, q, e_latent_loss


# speedup vs baseline: 1.0671x; 1.0221x over previous
import functools
import math

import jax
import jax.numpy as jnp
from jax.experimental import pallas as pl
from jax.experimental.pallas import tpu as pltpu
from jax.experimental.pallas import tpu_sc as plsc

J = 17
T = 34
H = 512
C = 1024
D = 512
BS = 256
BB = 8
G = BS // BB
R = BB * J
RT = BB * T
NZ = BS * T
DH = 32
EPS = 1e-5
SUB = 2

_GW = 16


def _ln(x, g, b):
    m = jnp.mean(x, -1, keepdims=True)
    v = jnp.mean((x - m) ** 2, -1, keepdims=True)
    return (x - m) / jnp.sqrt(v + EPS) * g + b


def _gelu(x):
    return x * 0.5 * (1.0 + jax.lax.erf(x * (1.0 / math.sqrt(2.0))))


def _enc_kernel(coords, w, inv, sw, sb, *rest):
    blocks = [rest[12 * k:12 * (k + 1)] for k in range(4)]
    lng, lnb, mt, mtb, few, feb, cbt = rest[48:55]
    z_ref, q_ref, cbsq_ref = rest[55:58]

    i = pl.program_id(0)

    @pl.when(i == 0)
    def _():
        cbsq_ref[...] = jnp.sum(cbt[...] * cbt[...], axis=0, keepdims=True)

    zs = []
    for hh_ in range(SUB):
        sl = pl.ds(hh_ * R, R)
        wv = w[sl, :]
        feat = jnp.dot(coords[sl, :], sw[...]) + sb[...]
        feat = feat * wv + inv[...] * (1.0 - wv)

        for (l1g, l1b, m1, t1b, m2, t2b, l2g, l2b,
             c1w, c1b, c2w, c2b) in blocks:
            y = _ln(feat, l1g[...], l1b[...])
            h = _gelu(jnp.dot(m1[...], y) + t1b[...])
            y = jnp.dot(m2[...], h) + t2b[...]
            zin = _ln(feat + y, l2g[...], l2b[...])
            hh = _gelu(jnp.dot(zin, c1w[...]) + c1b[...])
            zz = jnp.dot(hh, c2w[...]) + c2b[...]
            feat = feat + y + zz

        feat = _ln(feat, lng[...], lnb[...])
        tk = jnp.dot(mt[...], feat) + mtb[...]
        zs.append(jnp.dot(tk, few[...]) + feb[...])

    z = jnp.concatenate(zs, axis=0) if SUB > 1 else zs[0]
    zsq = jnp.sum(z * z, axis=1, keepdims=True)
    d2 = zsq - 2.0 * jnp.dot(z, cbt[...]) + cbsq_ref[...]
    dmin = jnp.min(d2, axis=1, keepdims=True)
    lanes = jax.lax.broadcasted_iota(jnp.int32, d2.shape, 1)
    q = jnp.min(jnp.where(d2 == dmin, lanes, C), axis=1, keepdims=True)

    z_ref[...] = z
    q_ref[...] = q


def _dec_kernel(z, zq, md, mdb, dsw, dsb,
                l1g, l1b, dm1, dt1b, dm2, dt2b, l2g, l2b,
                dc1w, dc1b, dc2w, dc2b,
                lng, lnb, rw, rb, rec_ref, lsum_ref):
    i = pl.program_id(0)
    zv = z[...]
    zqv = zq[...]

    @pl.when(i == 0)
    def _():
        lsum_ref[...] = jnp.zeros_like(lsum_ref)

    diff = zv - zqv
    lsum_ref[...] += jnp.sum(diff * diff, axis=(0, 1), keepdims=True)

    ste = zv + (zqv - zv)
    part = jnp.dot(md[...], ste) + mdb[...]
    dec = jnp.dot(part, dsw[...]) + dsb[...]

    y = _ln(dec, l1g[...], l1b[...])
    h = _gelu(jnp.dot(dm1[...], y) + dt1b[...])
    y = jnp.dot(dm2[...], h) + dt2b[...]
    zin = _ln(dec + y, l2g[...], l2b[...])
    hh = _gelu(jnp.dot(zin, dc1w[...]) + dc1b[...])
    zz = jnp.dot(hh, dc2w[...]) + dc2b[...]
    dec = dec + y + zz

    dec = _ln(dec, lng[...], lnb[...])
    rec_ref[...] = jnp.dot(dec, rw[...]) + rb[...]


def _const2(shape):
    return pl.BlockSpec(shape, lambda i: (0, 0))


_NW = 32
_BPW = NZ // _NW
_CH = 16
_K = 8


def _sc_gather(cb, q, n=NZ):
    mesh = plsc.VectorSubcoreMesh(core_axis_name="c", subcore_axis_name="s")
    grp = _K * _CH
    bpw = n // _NW
    assert bpw * _NW == n and bpw % 8 == 0

    groups = []
    off = 0
    while off < bpw:
        g = []
        while off < bpw and len(g) < _K:
            sz = min(_CH, bpw - off)
            g.append((off, sz))
            off += sz
        groups.append(g)

    @functools.partial(
        pl.kernel,
        out_type=jax.ShapeDtypeStruct((n, D), cb.dtype),
        mesh=mesh,
        scratch_types=[
            pltpu.VMEM((bpw,), jnp.int32),
            pltpu.VMEM((grp, D), jnp.float32),
            pltpu.SemaphoreType.DMA,
        ],
    )
    def kern(cb_hbm, q_hbm, o_hbm, idx_v, rows_v, sem):
        wid = jax.lax.axis_index("s") * 2 + jax.lax.axis_index("c")
        base = wid * bpw
        pltpu.sync_copy(q_hbm.at[pl.ds(base, bpw)], idx_v)
        for g in groups:
            cps = []
            g0 = g[0][0]
            for off, sz in g:
                cps.append(pltpu.async_copy(
                    cb_hbm.at[idx_v.at[pl.ds(off, sz)]],
                    rows_v.at[pl.ds(off - g0, sz)], sem))
            for cp in cps:
                cp.wait()
            gn = g[-1][0] + g[-1][1] - g0
            pltpu.sync_copy(rows_v.at[pl.ds(0, gn)],
                            o_hbm.at[pl.ds(base + g0, gn)])

    return kern(cb, q)


def _row(b):
    return b.reshape(1, -1)


def _bd(wt, bb=BB):
    return jnp.kron(jnp.eye(bb, dtype=wt.dtype), wt.T)


def _colb(b, bb=BB):
    return jnp.tile(b, bb).reshape(-1, 1)


def kernel(joints, joints_feature, cls_logits, params):
    del joints_feature, cls_logits
    coords = joints[:, :, :2].reshape(BS * J, 2)
    w2d = (joints[:, :, 2] != 0).astype(jnp.float32).reshape(BS * J, 1)

    enc_args = [coords, w2d,
                params["invisible_token"].reshape(1, H),
                params["start_embed"]["w"], _row(params["start_embed"]["b"])]
    enc_specs = [
        pl.BlockSpec((SUB * R, 2), lambda i: (i, 0)),
        pl.BlockSpec((SUB * R, 1), lambda i: (i, 0)),
        _const2((1, H)), _const2((2, H)), _const2((1, H)),
    ]
    for p in params["encoder"]:
        enc_args += [
            _row(p["ln1_g"]), _row(p["ln1_b"]),
            _bd(p["tok1"]["w"]), _colb(p["tok1"]["b"]),
            _bd(p["tok2"]["w"]), _colb(p["tok2"]["b"]),
            _row(p["ln2_g"]), _row(p["ln2_b"]),
            p["ch1"]["w"], _row(p["ch1"]["b"]),
            p["ch2"]["w"], _row(p["ch2"]["b"]),
        ]
        enc_specs += [
            _const2((1, H)), _const2((1, H)),
            _const2((BB * 64, R)), _const2((BB * 64, 1)),
            _const2((R, BB * 64)), _const2((R, 1)),
            _const2((1, H)), _const2((1, H)),
            _const2((H, H)), _const2((1, H)),
            _const2((H, H)), _const2((1, H)),
        ]
    enc_args += [
        _row(params["enc_ln_g"]), _row(params["enc_ln_b"]),
        _bd(params["token_mlp"]["w"]), _colb(params["token_mlp"]["b"]),
        params["feature_embed"]["w"], _row(params["feature_embed"]["b"]),
        params["codebook"].T,
    ]
    enc_specs += [
        _const2((1, H)), _const2((1, H)),
        _const2((RT, R)), _const2((RT, 1)),
        _const2((H, D)), _const2((1, D)),
        _const2((D, C)),
    ]

    NH = NZ // 2
    RH = BS * J // 2
    GH = G // 2

    def _enc_half(coords_h, w_h):
        return pl.pallas_call(
            _enc_kernel,
            grid=(GH // SUB,),
            in_specs=enc_specs,
            out_specs=[
                pl.BlockSpec((SUB * RT, D), lambda i: (i, 0)),
                pl.BlockSpec((SUB * RT, 1), lambda i: (i, 0)),
            ],
            out_shape=[
                jax.ShapeDtypeStruct((NH, D), jnp.float32),
                jax.ShapeDtypeStruct((NH, 1), jnp.int32),
            ],
            scratch_shapes=[pltpu.VMEM((1, C), jnp.float32)],
        )(coords_h, w_h, *enc_args[2:])

    z_h, q_h = [], []
    for h in range(2):
        zh, qh = _enc_half(coords[h * RH:(h + 1) * RH],
                           w2d[h * RH:(h + 1) * RH])
        z_h.append(zh)
        q_h.append(qh.reshape(NH))

    zq_h = [_sc_gather(params["codebook"], q_h[h], NH) for h in range(2)]

    dp = params["decoder"][0]
    dec_args = [
        _bd(params["decoder_token_mlp"]["w"]),
        _colb(params["decoder_token_mlp"]["b"]),
        params["decoder_start"]["w"], _row(params["decoder_start"]["b"]),
        _row(dp["ln1_g"]), _row(dp["ln1_b"]),
        _bd(dp["tok1"]["w"]), _colb(dp["tok1"]["b"]),
        _bd(dp["tok2"]["w"]), _colb(dp["tok2"]["b"]),
        _row(dp["ln2_g"]), _row(dp["ln2_b"]),
        dp["ch1"]["w"], _row(dp["ch1"]["b"]),
        dp["ch2"]["w"], _row(dp["ch2"]["b"]),
        _row(params["dec_ln_g"]), _row(params["dec_ln_b"]),
        params["recover_embed"]["w"], _row(params["recover_embed"]["b"]),
    ]
    dec_specs = [
        pl.BlockSpec((RT, D), lambda i: (i, 0)),
        pl.BlockSpec((RT, D), lambda i: (i, 0)),
        _const2((R, RT)), _const2((R, 1)),
        _const2((D, DH)), _const2((1, DH)),
        _const2((1, DH)), _const2((1, DH)),
        _const2((BB * 64, R)), _const2((BB * 64, 1)),
        _const2((R, BB * 64)), _const2((R, 1)),
        _const2((1, DH)), _const2((1, DH)),
        _const2((DH, 64)), _const2((1, 64)),
        _const2((64, DH)), _const2((1, DH)),
        _const2((1, DH)), _const2((1, DH)),
        _const2((DH, 2)), _const2((1, 2)),
    ]

    def _dec_half(zh, zqh):
        return pl.pallas_call(
            _dec_kernel,
            grid=(GH,),
            in_specs=dec_specs,
            out_specs=[
                pl.BlockSpec((R, 2), lambda i: (i, 0)),
                pl.BlockSpec((1, 1), lambda i: (0, 0)),
            ],
            out_shape=[
                jax.ShapeDtypeStruct((RH, 2), jnp.float32),
                jax.ShapeDtypeStruct((1, 1), jnp.float32),
            ],
        )(zh, zqh, *dec_args)

    rec_h, lsum_h = [], []
    for h in range(2):
        rh, lh = _dec_half(z_h[h], zq_h[h])
        rec_h.append(rh)
        lsum_h.append(lh)

    rec = jnp.concatenate(rec_h, axis=0).reshape(BS, J, 2)
    q = jnp.concatenate(q_h, axis=0)
    e_latent_loss = (lsum_h[0][0, 0] + lsum_h[1][0, 0]) / (NZ * D)
    return rec, q, e_latent_loss


# speedup vs baseline: 1.2151x; 1.1387x over previous
import functools
import math

import jax
import jax.numpy as jnp
from jax.experimental import pallas as pl
from jax.experimental.pallas import tpu as pltpu
from jax.experimental.pallas import tpu_sc as plsc

J = 17
T = 34
H = 512
C = 1024
D = 512
BS = 256
BB = 8
G = BS // BB
R = BB * J
RT = BB * T
NZ = BS * T
DH = 32
EPS = 1e-5
SUB = 2

_GW = 16


def _ln(x, g, b):
    m = jnp.mean(x, -1, keepdims=True)
    v = jnp.mean((x - m) ** 2, -1, keepdims=True)
    return (x - m) / jnp.sqrt(v + EPS) * g + b


def _gelu(x):
    return x * 0.5 * (1.0 + jax.lax.erf(x * (1.0 / math.sqrt(2.0))))


def _enc_kernel(coords, w, inv, sw, sb, *rest):
    blocks = [rest[12 * k:12 * (k + 1)] for k in range(4)]
    lng, lnb, mt, mtb, few, feb, cbt = rest[48:55]
    z_ref, q_ref, cbsq_ref = rest[55:58]

    i = pl.program_id(0)

    @pl.when(i == 0)
    def _():
        cbsq_ref[...] = jnp.sum(cbt[...] * cbt[...], axis=0, keepdims=True)

    zs = []
    for hh_ in range(SUB):
        sl = pl.ds(hh_ * R, R)
        wv = w[sl, :]
        feat = jnp.dot(coords[sl, :], sw[...]) + sb[...]
        feat = feat * wv + inv[...] * (1.0 - wv)

        for (l1g, l1b, m1, t1b, m2, t2b, l2g, l2b,
             c1w, c1b, c2w, c2b) in blocks:
            y = _ln(feat, l1g[...], l1b[...])
            h = _gelu(jnp.dot(m1[...], y) + t1b[...])
            y = jnp.dot(m2[...], h) + t2b[...]
            zin = _ln(feat + y, l2g[...], l2b[...])
            hh = _gelu(jnp.dot(zin, c1w[...]) + c1b[...])
            zz = jnp.dot(hh, c2w[...]) + c2b[...]
            feat = feat + y + zz

        feat = _ln(feat, lng[...], lnb[...])
        tk = jnp.dot(mt[...], feat) + mtb[...]
        zs.append(jnp.dot(tk, few[...]) + feb[...])

    z = jnp.concatenate(zs, axis=0) if SUB > 1 else zs[0]
    zsq = jnp.sum(z * z, axis=1, keepdims=True)
    d2 = zsq - 2.0 * jnp.dot(z, cbt[...]) + cbsq_ref[...]
    dmin = jnp.min(d2, axis=1, keepdims=True)
    lanes = jax.lax.broadcasted_iota(jnp.int32, d2.shape, 1)
    q = jnp.min(jnp.where(d2 == dmin, lanes, C), axis=1, keepdims=True)

    z_ref[...] = z
    q_ref[...] = q


def _dec_body(zv, zqv, md, mdb, dsw, dsb,
              l1g, l1b, dm1, dt1b, dm2, dt2b, l2g, l2b,
              dc1w, dc1b, dc2w, dc2b,
              lng, lnb, rw, rb, rec_ref, lsum_ref):
    i = pl.program_id(0)

    @pl.when(i == 0)
    def _():
        lsum_ref[...] = jnp.zeros_like(lsum_ref)

    diff = zv - zqv
    lsum_ref[...] += jnp.sum(diff * diff, axis=(0, 1), keepdims=True)

    ste = zv + (zqv - zv)
    part = jnp.dot(md[...], ste) + mdb[...]
    dec = jnp.dot(part, dsw[...]) + dsb[...]

    y = _ln(dec, l1g[...], l1b[...])
    h = _gelu(jnp.dot(dm1[...], y) + dt1b[...])
    y = jnp.dot(dm2[...], h) + dt2b[...]
    zin = _ln(dec + y, l2g[...], l2b[...])
    hh = _gelu(jnp.dot(zin, dc1w[...]) + dc1b[...])
    zz = jnp.dot(hh, dc2w[...]) + dc2b[...]
    dec = dec + y + zz

    dec = _ln(dec, lng[...], lnb[...])
    rec_ref[...] = jnp.dot(dec, rw[...]) + rb[...]


def _dec_kernel(z, zq, *args):
    _dec_body(z[...], zq[...], *args)


def _dec_oh_kernel(z, q, cb, *args):
    lanes = jax.lax.broadcasted_iota(jnp.int32, (q.shape[0], C), 1)
    oh = (lanes == q[...]).astype(jnp.float32)
    zqv = jnp.dot(oh, cb[...])
    _dec_body(z[...], zqv, *args)


def _const2(shape):
    return pl.BlockSpec(shape, lambda i: (0, 0))


_NW = 32
_BPW = NZ // _NW
_CH = 16
_K = 8


def _sc_gather(cb, q, n=NZ):
    mesh = plsc.VectorSubcoreMesh(core_axis_name="c", subcore_axis_name="s")
    grp = _K * _CH
    bpw = n // _NW
    assert bpw * _NW == n and bpw % 8 == 0

    groups = []
    off = 0
    while off < bpw:
        g = []
        while off < bpw and len(g) < _K:
            sz = min(_CH, bpw - off)
            g.append((off, sz))
            off += sz
        groups.append(g)

    @functools.partial(
        pl.kernel,
        out_type=jax.ShapeDtypeStruct((n, D), cb.dtype),
        mesh=mesh,
        scratch_types=[
            pltpu.VMEM((bpw,), jnp.int32),
            pltpu.VMEM((grp, D), jnp.float32),
            pltpu.SemaphoreType.DMA,
        ],
    )
    def kern(cb_hbm, q_hbm, o_hbm, idx_v, rows_v, sem):
        wid = jax.lax.axis_index("s") * 2 + jax.lax.axis_index("c")
        base = wid * bpw
        pltpu.sync_copy(q_hbm.at[pl.ds(base, bpw)], idx_v)
        for g in groups:
            cps = []
            g0 = g[0][0]
            for off, sz in g:
                cps.append(pltpu.async_copy(
                    cb_hbm.at[idx_v.at[pl.ds(off, sz)]],
                    rows_v.at[pl.ds(off - g0, sz)], sem))
            for cp in cps:
                cp.wait()
            gn = g[-1][0] + g[-1][1] - g0
            pltpu.sync_copy(rows_v.at[pl.ds(0, gn)],
                            o_hbm.at[pl.ds(base + g0, gn)])

    return kern(cb, q)


def _row(b):
    return b.reshape(1, -1)


def _bd(wt, bb=BB):
    return jnp.kron(jnp.eye(bb, dtype=wt.dtype), wt.T)


def _colb(b, bb=BB):
    return jnp.tile(b, bb).reshape(-1, 1)


def kernel(joints, joints_feature, cls_logits, params):
    del joints_feature, cls_logits
    coords = joints[:, :, :2].reshape(BS * J, 2)
    w2d = (joints[:, :, 2] != 0).astype(jnp.float32).reshape(BS * J, 1)

    enc_args = [coords, w2d,
                params["invisible_token"].reshape(1, H),
                params["start_embed"]["w"], _row(params["start_embed"]["b"])]
    enc_specs = [
        pl.BlockSpec((SUB * R, 2), lambda i: (i, 0)),
        pl.BlockSpec((SUB * R, 1), lambda i: (i, 0)),
        _const2((1, H)), _const2((2, H)), _const2((1, H)),
    ]
    for p in params["encoder"]:
        enc_args += [
            _row(p["ln1_g"]), _row(p["ln1_b"]),
            _bd(p["tok1"]["w"]), _colb(p["tok1"]["b"]),
            _bd(p["tok2"]["w"]), _colb(p["tok2"]["b"]),
            _row(p["ln2_g"]), _row(p["ln2_b"]),
            p["ch1"]["w"], _row(p["ch1"]["b"]),
            p["ch2"]["w"], _row(p["ch2"]["b"]),
        ]
        enc_specs += [
            _const2((1, H)), _const2((1, H)),
            _const2((BB * 64, R)), _const2((BB * 64, 1)),
            _const2((R, BB * 64)), _const2((R, 1)),
            _const2((1, H)), _const2((1, H)),
            _const2((H, H)), _const2((1, H)),
            _const2((H, H)), _const2((1, H)),
        ]
    enc_args += [
        _row(params["enc_ln_g"]), _row(params["enc_ln_b"]),
        _bd(params["token_mlp"]["w"]), _colb(params["token_mlp"]["b"]),
        params["feature_embed"]["w"], _row(params["feature_embed"]["b"]),
        params["codebook"].T,
    ]
    enc_specs += [
        _const2((1, H)), _const2((1, H)),
        _const2((RT, R)), _const2((RT, 1)),
        _const2((H, D)), _const2((1, D)),
        _const2((D, C)),
    ]

    NH = NZ // 2
    RH = BS * J // 2
    GH = G // 2

    def _enc_half(coords_h, w_h):
        return pl.pallas_call(
            _enc_kernel,
            grid=(GH // SUB,),
            in_specs=enc_specs,
            out_specs=[
                pl.BlockSpec((SUB * RT, D), lambda i: (i, 0)),
                pl.BlockSpec((SUB * RT, 1), lambda i: (i, 0)),
            ],
            out_shape=[
                jax.ShapeDtypeStruct((NH, D), jnp.float32),
                jax.ShapeDtypeStruct((NH, 1), jnp.int32),
            ],
            scratch_shapes=[pltpu.VMEM((1, C), jnp.float32)],
        )(coords_h, w_h, *enc_args[2:])

    z_h, q_h, q2d_h = [], [], []
    for h in range(2):
        zh, qh = _enc_half(coords[h * RH:(h + 1) * RH],
                           w2d[h * RH:(h + 1) * RH])
        z_h.append(zh)
        q2d_h.append(qh)
        q_h.append(qh.reshape(NH))

    zq0 = _sc_gather(params["codebook"], q_h[0], NH)

    dp = params["decoder"][0]
    dec_args = [
        _bd(params["decoder_token_mlp"]["w"]),
        _colb(params["decoder_token_mlp"]["b"]),
        params["decoder_start"]["w"], _row(params["decoder_start"]["b"]),
        _row(dp["ln1_g"]), _row(dp["ln1_b"]),
        _bd(dp["tok1"]["w"]), _colb(dp["tok1"]["b"]),
        _bd(dp["tok2"]["w"]), _colb(dp["tok2"]["b"]),
        _row(dp["ln2_g"]), _row(dp["ln2_b"]),
        dp["ch1"]["w"], _row(dp["ch1"]["b"]),
        dp["ch2"]["w"], _row(dp["ch2"]["b"]),
        _row(params["dec_ln_g"]), _row(params["dec_ln_b"]),
        params["recover_embed"]["w"], _row(params["recover_embed"]["b"]),
    ]
    dec_specs = [
        pl.BlockSpec((RT, D), lambda i: (i, 0)),
        pl.BlockSpec((RT, D), lambda i: (i, 0)),
        _const2((R, RT)), _const2((R, 1)),
        _const2((D, DH)), _const2((1, DH)),
        _const2((1, DH)), _const2((1, DH)),
        _const2((BB * 64, R)), _const2((BB * 64, 1)),
        _const2((R, BB * 64)), _const2((R, 1)),
        _const2((1, DH)), _const2((1, DH)),
        _const2((DH, 64)), _const2((1, 64)),
        _const2((64, DH)), _const2((1, DH)),
        _const2((1, DH)), _const2((1, DH)),
        _const2((DH, 2)), _const2((1, 2)),
    ]

    dec_outs = [
        pl.BlockSpec((R, 2), lambda i: (i, 0)),
        pl.BlockSpec((1, 1), lambda i: (0, 0)),
    ]
    dec_types = [
        jax.ShapeDtypeStruct((RH, 2), jnp.float32),
        jax.ShapeDtypeStruct((1, 1), jnp.float32),
    ]

    rec1, lsum1 = pl.pallas_call(
        _dec_oh_kernel,
        grid=(GH,),
        in_specs=[dec_specs[0],
                  pl.BlockSpec((RT, 1), lambda i: (i, 0)),
                  _const2((C, D))] + dec_specs[2:],
        out_specs=dec_outs,
        out_shape=dec_types,
    )(z_h[1], q2d_h[1], params["codebook"], *dec_args)

    rec0, lsum0 = pl.pallas_call(
        _dec_kernel,
        grid=(GH,),
        in_specs=dec_specs,
        out_specs=dec_outs,
        out_shape=dec_types,
    )(z_h[0], zq0, *dec_args)

    rec_h = [rec0, rec1]
    lsum_h = [lsum0, lsum1]

    rec = jnp.concatenate(rec_h, axis=0).reshape(BS, J, 2)
    q = jnp.concatenate(q_h, axis=0)
    e_latent_loss = (lsum_h[0][0, 0] + lsum_h[1][0, 0]) / (NZ * D)
    return rec, q, e_latent_loss


# speedup vs baseline: 1.4824x; 1.2200x over previous
import functools
import math

import jax
import jax.numpy as jnp
from jax.experimental import pallas as pl
from jax.experimental.pallas import tpu as pltpu
from jax.experimental.pallas import tpu_sc as plsc

J = 17
T = 34
H = 512
C = 1024
D = 512
BS = 256
BB = 8
G = BS // BB
R = BB * J
RT = BB * T
NZ = BS * T
DH = 32
EPS = 1e-5
SUB = 2

_GW = 16


def _ln(x, g, b):
    m = jnp.mean(x, -1, keepdims=True)
    v = jnp.mean((x - m) ** 2, -1, keepdims=True)
    return (x - m) / jnp.sqrt(v + EPS) * g + b


def _gelu(x):
    return x * 0.5 * (1.0 + jax.lax.erf(x * (1.0 / math.sqrt(2.0))))


def _enc_kernel(coords, w, inv, sw, sb, *rest):
    blocks = [rest[12 * k:12 * (k + 1)] for k in range(4)]
    lng, lnb, mt, mtb, few, feb, cbt = rest[48:55]
    z_ref, q_ref, cbsq_ref = rest[55:58]

    i = pl.program_id(0)

    @pl.when(i == 0)
    def _():
        cbsq_ref[...] = jnp.sum(cbt[...] * cbt[...], axis=0, keepdims=True)

    feats = []
    for hh_ in range(SUB):
        sl = pl.ds(hh_ * R, R)
        wv = w[sl, :]
        feat = jnp.dot(coords[sl, :], sw[...]) + sb[...]
        feats.append(feat * wv + inv[...] * (1.0 - wv))

    for (l1g, l1b, m1, t1b, m2, t2b, l2g, l2b,
         c1w, c1b, c2w, c2b) in blocks:
        ys = [_ln(f, l1g[...], l1b[...]) for f in feats]
        hs = [_gelu(jnp.dot(m1[...], y) + t1b[...]) for y in ys]
        ys = [jnp.dot(m2[...], h) + t2b[...] for h in hs]
        zins = [_ln(f + y, l2g[...], l2b[...]) for f, y in zip(feats, ys)]
        hhs = [_gelu(jnp.dot(zin, c1w[...]) + c1b[...]) for zin in zins]
        zzs = [jnp.dot(hh, c2w[...]) + c2b[...] for hh in hhs]
        feats = [f + y + zz for f, y, zz in zip(feats, ys, zzs)]

    feats = [_ln(f, lng[...], lnb[...]) for f in feats]
    tks = [jnp.dot(mt[...], f) + mtb[...] for f in feats]
    zs = [jnp.dot(tk, few[...]) + feb[...] for tk in tks]

    z = jnp.concatenate(zs, axis=0) if SUB > 1 else zs[0]
    zsq = jnp.sum(z * z, axis=1, keepdims=True)
    d2 = zsq - 2.0 * jnp.dot(z, cbt[...]) + cbsq_ref[...]
    dmin = jnp.min(d2, axis=1, keepdims=True)
    lanes = jax.lax.broadcasted_iota(jnp.int32, d2.shape, 1)
    q = jnp.min(jnp.where(d2 == dmin, lanes, C), axis=1, keepdims=True)

    z_ref[...] = z
    q_ref[...] = q


def _dec_body(zv, zqv, md, mdb, dsw, dsb,
              l1g, l1b, dm1, dt1b, dm2, dt2b, l2g, l2b,
              dc1w, dc1b, dc2w, dc2b,
              lng, lnb, rw, rb, rec_ref, lsum_ref):
    i = pl.program_id(0)

    @pl.when(i == 0)
    def _():
        lsum_ref[...] = jnp.zeros_like(lsum_ref)

    diff = zv - zqv
    lsum_ref[...] += jnp.sum(diff * diff, axis=(0, 1), keepdims=True)

    ste = zv + (zqv - zv)
    part = jnp.dot(md[...], ste) + mdb[...]
    dec = jnp.dot(part, dsw[...]) + dsb[...]

    y = _ln(dec, l1g[...], l1b[...])
    h = _gelu(jnp.dot(dm1[...], y) + dt1b[...])
    y = jnp.dot(dm2[...], h) + dt2b[...]
    zin = _ln(dec + y, l2g[...], l2b[...])
    hh = _gelu(jnp.dot(zin, dc1w[...]) + dc1b[...])
    zz = jnp.dot(hh, dc2w[...]) + dc2b[...]
    dec = dec + y + zz

    dec = _ln(dec, lng[...], lnb[...])
    rec_ref[...] = jnp.dot(dec, rw[...]) + rb[...]


def _dec_kernel(z, zq, *args):
    _dec_body(z[...], zq[...], *args)


def _dec_oh_kernel(z, q, cb, *args):
    lanes = jax.lax.broadcasted_iota(jnp.int32, (q.shape[0], C), 1)
    oh = (lanes == q[...]).astype(jnp.float32)
    zqv = jnp.dot(oh, cb[...])
    _dec_body(z[...], zqv, *args)


def _const2(shape):
    return pl.BlockSpec(shape, lambda i: (0, 0))


_NW = 32
_BPW = NZ // _NW
_CH = 16
_K = 8


def _sc_gather(cb, q, n=NZ):
    mesh = plsc.VectorSubcoreMesh(core_axis_name="c", subcore_axis_name="s")
    grp = _K * _CH
    bpw = n // _NW
    assert bpw * _NW == n and bpw % 8 == 0

    groups = []
    off = 0
    while off < bpw:
        g = []
        while off < bpw and len(g) < _K:
            sz = min(_CH, bpw - off)
            g.append((off, sz))
            off += sz
        groups.append(g)

    @functools.partial(
        pl.kernel,
        out_type=jax.ShapeDtypeStruct((n, D), cb.dtype),
        mesh=mesh,
        scratch_types=[
            pltpu.VMEM((bpw,), jnp.int32),
            pltpu.VMEM((grp, D), jnp.float32),
            pltpu.SemaphoreType.DMA,
        ],
    )
    def kern(cb_hbm, q_hbm, o_hbm, idx_v, rows_v, sem):
        wid = jax.lax.axis_index("s") * 2 + jax.lax.axis_index("c")
        base = wid * bpw
        pltpu.sync_copy(q_hbm.at[pl.ds(base, bpw)], idx_v)
        for g in groups:
            cps = []
            g0 = g[0][0]
            for off, sz in g:
                cps.append(pltpu.async_copy(
                    cb_hbm.at[idx_v.at[pl.ds(off, sz)]],
                    rows_v.at[pl.ds(off - g0, sz)], sem))
            for cp in cps:
                cp.wait()
            gn = g[-1][0] + g[-1][1] - g0
            pltpu.sync_copy(rows_v.at[pl.ds(0, gn)],
                            o_hbm.at[pl.ds(base + g0, gn)])

    return kern(cb, q)


def _row(b):
    return b.reshape(1, -1)


def _bd(wt, bb=BB):
    return jnp.kron(jnp.eye(bb, dtype=wt.dtype), wt.T)


def _colb(b, bb=BB):
    return jnp.tile(b, bb).reshape(-1, 1)


def kernel(joints, joints_feature, cls_logits, params):
    del joints_feature, cls_logits
    coords = joints[:, :, :2].reshape(BS * J, 2)
    w2d = (joints[:, :, 2] != 0).astype(jnp.float32).reshape(BS * J, 1)

    enc_args = [coords, w2d,
                params["invisible_token"].reshape(1, H),
                params["start_embed"]["w"], _row(params["start_embed"]["b"])]
    enc_specs = [
        pl.BlockSpec((SUB * R, 2), lambda i: (i, 0)),
        pl.BlockSpec((SUB * R, 1), lambda i: (i, 0)),
        _const2((1, H)), _const2((2, H)), _const2((1, H)),
    ]
    for p in params["encoder"]:
        enc_args += [
            _row(p["ln1_g"]), _row(p["ln1_b"]),
            _bd(p["tok1"]["w"]), _colb(p["tok1"]["b"]),
            _bd(p["tok2"]["w"]), _colb(p["tok2"]["b"]),
            _row(p["ln2_g"]), _row(p["ln2_b"]),
            p["ch1"]["w"], _row(p["ch1"]["b"]),
            p["ch2"]["w"], _row(p["ch2"]["b"]),
        ]
        enc_specs += [
            _const2((1, H)), _const2((1, H)),
            _const2((BB * 64, R)), _const2((BB * 64, 1)),
            _const2((R, BB * 64)), _const2((R, 1)),
            _const2((1, H)), _const2((1, H)),
            _const2((H, H)), _const2((1, H)),
            _const2((H, H)), _const2((1, H)),
        ]
    enc_args += [
        _row(params["enc_ln_g"]), _row(params["enc_ln_b"]),
        _bd(params["token_mlp"]["w"]), _colb(params["token_mlp"]["b"]),
        params["feature_embed"]["w"], _row(params["feature_embed"]["b"]),
        params["codebook"].T,
    ]
    enc_specs += [
        _const2((1, H)), _const2((1, H)),
        _const2((RT, R)), _const2((RT, 1)),
        _const2((H, D)), _const2((1, D)),
        _const2((D, C)),
    ]

    NH = NZ // 2
    RH = BS * J // 2
    GH = G // 2

    def _enc_half(coords_h, w_h):
        return pl.pallas_call(
            _enc_kernel,
            grid=(GH // SUB,),
            in_specs=enc_specs,
            out_specs=[
                pl.BlockSpec((SUB * RT, D), lambda i: (i, 0)),
                pl.BlockSpec((SUB * RT, 1), lambda i: (i, 0)),
            ],
            out_shape=[
                jax.ShapeDtypeStruct((NH, D), jnp.float32),
                jax.ShapeDtypeStruct((NH, 1), jnp.int32),
            ],
            scratch_shapes=[pltpu.VMEM((1, C), jnp.float32)],
        )(coords_h, w_h, *enc_args[2:])

    z_h, q_h, q2d_h = [], [], []
    for h in range(2):
        zh, qh = _enc_half(coords[h * RH:(h + 1) * RH],
                           w2d[h * RH:(h + 1) * RH])
        z_h.append(zh)
        q2d_h.append(qh)
        q_h.append(qh.reshape(NH))

    zq0 = _sc_gather(params["codebook"], q_h[0], NH)

    dp = params["decoder"][0]
    dec_args = [
        _bd(params["decoder_token_mlp"]["w"]),
        _colb(params["decoder_token_mlp"]["b"]),
        params["decoder_start"]["w"], _row(params["decoder_start"]["b"]),
        _row(dp["ln1_g"]), _row(dp["ln1_b"]),
        _bd(dp["tok1"]["w"]), _colb(dp["tok1"]["b"]),
        _bd(dp["tok2"]["w"]), _colb(dp["tok2"]["b"]),
        _row(dp["ln2_g"]), _row(dp["ln2_b"]),
        dp["ch1"]["w"], _row(dp["ch1"]["b"]),
        dp["ch2"]["w"], _row(dp["ch2"]["b"]),
        _row(params["dec_ln_g"]), _row(params["dec_ln_b"]),
        params["recover_embed"]["w"], _row(params["recover_embed"]["b"]),
    ]
    dec_specs = [
        pl.BlockSpec((RT, D), lambda i: (i, 0)),
        pl.BlockSpec((RT, D), lambda i: (i, 0)),
        _const2((R, RT)), _const2((R, 1)),
        _const2((D, DH)), _const2((1, DH)),
        _const2((1, DH)), _const2((1, DH)),
        _const2((BB * 64, R)), _const2((BB * 64, 1)),
        _const2((R, BB * 64)), _const2((R, 1)),
        _const2((1, DH)), _const2((1, DH)),
        _const2((DH, 64)), _const2((1, 64)),
        _const2((64, DH)), _const2((1, DH)),
        _const2((1, DH)), _const2((1, DH)),
        _const2((DH, 2)), _const2((1, 2)),
    ]

    dec_outs = [
        pl.BlockSpec((R, 2), lambda i: (i, 0)),
        pl.BlockSpec((1, 1), lambda i: (0, 0)),
    ]
    dec_types = [
        jax.ShapeDtypeStruct((RH, 2), jnp.float32),
        jax.ShapeDtypeStruct((1, 1), jnp.float32),
    ]

    rec1, lsum1 = pl.pallas_call(
        _dec_oh_kernel,
        grid=(GH,),
        in_specs=[dec_specs[0],
                  pl.BlockSpec((RT, 1), lambda i: (i, 0)),
                  _const2((C, D))] + dec_specs[2:],
        out_specs=dec_outs,
        out_shape=dec_types,
    )(z_h[1], q2d_h[1], params["codebook"], *dec_args)

    rec0, lsum0 = pl.pallas_call(
        _dec_kernel,
        grid=(GH,),
        in_specs=dec_specs,
        out_specs=dec_outs,
        out_shape=dec_types,
    )(z_h[0], zq0, *dec_args)

    rec_h = [rec0, rec1]
    lsum_h = [lsum0, lsum1]

    rec = jnp.concatenate(rec_h, axis=0).reshape(BS, J, 2)
    q = jnp.concatenate(q_h, axis=0)
    e_latent_loss = (lsum_h[0][0, 0] + lsum_h[1][0, 0]) / (NZ * D)
    return rec, q, e_latent_loss


# speedup vs baseline: 1.6001x; 1.0794x over previous
import functools
import math

import jax
import jax.numpy as jnp
from jax.experimental import pallas as pl
from jax.experimental.pallas import tpu as pltpu
from jax.experimental.pallas import tpu_sc as plsc

J = 17
T = 34
H = 512
C = 1024
D = 512
BS = 256
BB = 8
G = BS // BB
R = BB * J
RT = BB * T
NZ = BS * T
DH = 32
EPS = 1e-5
SUB = 2

_GW = 16


def _ln(x, g, b):
    m = jnp.mean(x, -1, keepdims=True)
    v = jnp.mean((x - m) ** 2, -1, keepdims=True)
    return (x - m) / jnp.sqrt(v + EPS) * g + b


def _gelu(x):
    return x * 0.5 * (1.0 + jax.lax.erf(x * (1.0 / math.sqrt(2.0))))


def _enc_kernel(coords, w, inv, sw, sb, *rest):
    blocks = [rest[12 * k:12 * (k + 1)] for k in range(4)]
    lng, lnb, mt, mtb, few, feb, cbt = rest[48:55]
    z_ref, q_ref, cbsq_ref = rest[55:58]

    i = pl.program_id(0)

    @pl.when(i == 0)
    def _():
        cbsq_ref[...] = jnp.sum(cbt[...] * cbt[...], axis=0, keepdims=True)

    feats = []
    for hh_ in range(SUB):
        sl = pl.ds(hh_ * R, R)
        wv = w[sl, :]
        feat = jnp.dot(coords[sl, :], sw[...]) + sb[...]
        feats.append(feat * wv + inv[...] * (1.0 - wv))

    for (l1g, l1b, m1, t1b, m2, t2b, l2g, l2b,
         c1w, c1b, c2w, c2b) in blocks:
        ys = [_ln(f, l1g[...], l1b[...]) for f in feats]
        hs = [_gelu(jnp.dot(m1[...], y) + t1b[...]) for y in ys]
        ys = [jnp.dot(m2[...], h) + t2b[...] for h in hs]
        zins = [_ln(f + y, l2g[...], l2b[...]) for f, y in zip(feats, ys)]
        hhs = [_gelu(jnp.dot(zin, c1w[...]) + c1b[...]) for zin in zins]
        zzs = [jnp.dot(hh, c2w[...]) + c2b[...] for hh in hhs]
        feats = [f + y + zz for f, y, zz in zip(feats, ys, zzs)]

    feats = [_ln(f, lng[...], lnb[...]) for f in feats]
    tks = [jnp.dot(mt[...], f) + mtb[...] for f in feats]
    zs = [jnp.dot(tk, few[...]) + feb[...] for tk in tks]

    z = jnp.concatenate(zs, axis=0) if SUB > 1 else zs[0]
    zsq = jnp.sum(z * z, axis=1, keepdims=True)
    d2 = zsq - 2.0 * jnp.dot(z, cbt[...]) + cbsq_ref[...]
    dmin = jnp.min(d2, axis=1, keepdims=True)
    lanes = jax.lax.broadcasted_iota(jnp.int32, d2.shape, 1)
    q = jnp.min(jnp.where(d2 == dmin, lanes, C), axis=1, keepdims=True)

    z_ref[...] = z
    q_ref[...] = q


def _dec_body(zvs, zqvs, md, mdb, dsw, dsb,
              l1g, l1b, dm1, dt1b, dm2, dt2b, l2g, l2b,
              dc1w, dc1b, dc2w, dc2b,
              lng, lnb, rw, rb, rec_ref, lsum_ref):
    i = pl.program_id(0)

    @pl.when(i == 0)
    def _():
        lsum_ref[...] = jnp.zeros_like(lsum_ref)

    acc = None
    for zv, zqv in zip(zvs, zqvs):
        diff = zv - zqv
        p = jnp.sum(diff * diff, axis=(0, 1), keepdims=True)
        acc = p if acc is None else acc + p
    lsum_ref[...] += acc

    decs = [jnp.dot(jnp.dot(md[...], zv + (zqv - zv)) + mdb[...],
                    dsw[...]) + dsb[...]
            for zv, zqv in zip(zvs, zqvs)]

    ys = [_ln(d, l1g[...], l1b[...]) for d in decs]
    hs = [_gelu(jnp.dot(dm1[...], y) + dt1b[...]) for y in ys]
    ys = [jnp.dot(dm2[...], h) + dt2b[...] for h in hs]
    zins = [_ln(d + y, l2g[...], l2b[...]) for d, y in zip(decs, ys)]
    hhs = [_gelu(jnp.dot(zin, dc1w[...]) + dc1b[...]) for zin in zins]
    zzs = [jnp.dot(hh, dc2w[...]) + dc2b[...] for hh in hhs]
    decs = [d + y + zz for d, y, zz in zip(decs, ys, zzs)]

    decs = [_ln(d, lng[...], lnb[...]) for d in decs]
    for c, d in enumerate(decs):
        rec_ref[pl.ds(c * R, R), :] = jnp.dot(d, rw[...]) + rb[...]


def _split(x):
    return [x[c * RT:(c + 1) * RT, :] for c in range(SUB)]


def _dec_kernel(z, zq, *args):
    _dec_body(_split(z[...]), _split(zq[...]), *args)


def _dec_oh_kernel(z, q, cb, *args):
    lanes = jax.lax.broadcasted_iota(jnp.int32, (SUB * RT, C), 1)
    oh = (lanes == q[...]).astype(jnp.float32)
    zqv = jnp.dot(oh, cb[...])
    _dec_body(_split(z[...]), _split(zqv), *args)


def _const2(shape):
    return pl.BlockSpec(shape, lambda i: (0, 0))


_NW = 32
_BPW = NZ // _NW
_CH = 16
_K = 8


def _sc_gather(cb, q, n=NZ):
    mesh = plsc.VectorSubcoreMesh(core_axis_name="c", subcore_axis_name="s")
    grp = _K * _CH
    bpw = n // _NW
    assert bpw * _NW == n and bpw % 8 == 0

    groups = []
    off = 0
    while off < bpw:
        g = []
        while off < bpw and len(g) < _K:
            sz = min(_CH, bpw - off)
            g.append((off, sz))
            off += sz
        groups.append(g)

    @functools.partial(
        pl.kernel,
        out_type=jax.ShapeDtypeStruct((n, D), cb.dtype),
        mesh=mesh,
        scratch_types=[
            pltpu.VMEM((bpw,), jnp.int32),
            pltpu.VMEM((grp, D), jnp.float32),
            pltpu.SemaphoreType.DMA,
        ],
    )
    def kern(cb_hbm, q_hbm, o_hbm, idx_v, rows_v, sem):
        wid = jax.lax.axis_index("s") * 2 + jax.lax.axis_index("c")
        base = wid * bpw
        pltpu.sync_copy(q_hbm.at[pl.ds(base, bpw)], idx_v)
        for g in groups:
            cps = []
            g0 = g[0][0]
            for off, sz in g:
                cps.append(pltpu.async_copy(
                    cb_hbm.at[idx_v.at[pl.ds(off, sz)]],
                    rows_v.at[pl.ds(off - g0, sz)], sem))
            for cp in cps:
                cp.wait()
            gn = g[-1][0] + g[-1][1] - g0
            pltpu.sync_copy(rows_v.at[pl.ds(0, gn)],
                            o_hbm.at[pl.ds(base + g0, gn)])

    return kern(cb, q)


def _row(b):
    return b.reshape(1, -1)


def _bd(wt, bb=BB):
    return jnp.kron(jnp.eye(bb, dtype=wt.dtype), wt.T)


def _colb(b, bb=BB):
    return jnp.tile(b, bb).reshape(-1, 1)


def kernel(joints, joints_feature, cls_logits, params):
    del joints_feature, cls_logits
    coords = joints[:, :, :2].reshape(BS * J, 2)
    w2d = (joints[:, :, 2] != 0).astype(jnp.float32).reshape(BS * J, 1)

    enc_args = [coords, w2d,
                params["invisible_token"].reshape(1, H),
                params["start_embed"]["w"], _row(params["start_embed"]["b"])]
    enc_specs = [
        pl.BlockSpec((SUB * R, 2), lambda i: (i, 0)),
        pl.BlockSpec((SUB * R, 1), lambda i: (i, 0)),
        _const2((1, H)), _const2((2, H)), _const2((1, H)),
    ]
    for p in params["encoder"]:
        enc_args += [
            _row(p["ln1_g"]), _row(p["ln1_b"]),
            _bd(p["tok1"]["w"]), _colb(p["tok1"]["b"]),
            _bd(p["tok2"]["w"]), _colb(p["tok2"]["b"]),
            _row(p["ln2_g"]), _row(p["ln2_b"]),
            p["ch1"]["w"], _row(p["ch1"]["b"]),
            p["ch2"]["w"], _row(p["ch2"]["b"]),
        ]
        enc_specs += [
            _const2((1, H)), _const2((1, H)),
            _const2((BB * 64, R)), _const2((BB * 64, 1)),
            _const2((R, BB * 64)), _const2((R, 1)),
            _const2((1, H)), _const2((1, H)),
            _const2((H, H)), _const2((1, H)),
            _const2((H, H)), _const2((1, H)),
        ]
    enc_args += [
        _row(params["enc_ln_g"]), _row(params["enc_ln_b"]),
        _bd(params["token_mlp"]["w"]), _colb(params["token_mlp"]["b"]),
        params["feature_embed"]["w"], _row(params["feature_embed"]["b"]),
        params["codebook"].T,
    ]
    enc_specs += [
        _const2((1, H)), _const2((1, H)),
        _const2((RT, R)), _const2((RT, 1)),
        _const2((H, D)), _const2((1, D)),
        _const2((D, C)),
    ]

    NH = NZ // 2
    RH = BS * J // 2
    GH = G // 2

    def _enc_half(coords_h, w_h):
        return pl.pallas_call(
            _enc_kernel,
            grid=(GH // SUB,),
            in_specs=enc_specs,
            out_specs=[
                pl.BlockSpec((SUB * RT, D), lambda i: (i, 0)),
                pl.BlockSpec((SUB * RT, 1), lambda i: (i, 0)),
            ],
            out_shape=[
                jax.ShapeDtypeStruct((NH, D), jnp.float32),
                jax.ShapeDtypeStruct((NH, 1), jnp.int32),
            ],
            scratch_shapes=[pltpu.VMEM((1, C), jnp.float32)],
        )(coords_h, w_h, *enc_args[2:])

    z_h, q_h, q2d_h = [], [], []
    for h in range(2):
        zh, qh = _enc_half(coords[h * RH:(h + 1) * RH],
                           w2d[h * RH:(h + 1) * RH])
        z_h.append(zh)
        q2d_h.append(qh)
        q_h.append(qh.reshape(NH))

    zq0 = _sc_gather(params["codebook"], q_h[0], NH)

    dp = params["decoder"][0]
    dec_args = [
        _bd(params["decoder_token_mlp"]["w"]),
        _colb(params["decoder_token_mlp"]["b"]),
        params["decoder_start"]["w"], _row(params["decoder_start"]["b"]),
        _row(dp["ln1_g"]), _row(dp["ln1_b"]),
        _bd(dp["tok1"]["w"]), _colb(dp["tok1"]["b"]),
        _bd(dp["tok2"]["w"]), _colb(dp["tok2"]["b"]),
        _row(dp["ln2_g"]), _row(dp["ln2_b"]),
        dp["ch1"]["w"], _row(dp["ch1"]["b"]),
        dp["ch2"]["w"], _row(dp["ch2"]["b"]),
        _row(params["dec_ln_g"]), _row(params["dec_ln_b"]),
        params["recover_embed"]["w"], _row(params["recover_embed"]["b"]),
    ]
    dec_specs = [
        pl.BlockSpec((SUB * RT, D), lambda i: (i, 0)),
        pl.BlockSpec((SUB * RT, D), lambda i: (i, 0)),
        _const2((R, RT)), _const2((R, 1)),
        _const2((D, DH)), _const2((1, DH)),
        _const2((1, DH)), _const2((1, DH)),
        _const2((BB * 64, R)), _const2((BB * 64, 1)),
        _const2((R, BB * 64)), _const2((R, 1)),
        _const2((1, DH)), _const2((1, DH)),
        _const2((DH, 64)), _const2((1, 64)),
        _const2((64, DH)), _const2((1, DH)),
        _const2((1, DH)), _const2((1, DH)),
        _const2((DH, 2)), _const2((1, 2)),
    ]

    dec_outs = [
        pl.BlockSpec((SUB * R, 2), lambda i: (i, 0)),
        pl.BlockSpec((1, 1), lambda i: (0, 0)),
    ]
    dec_types = [
        jax.ShapeDtypeStruct((RH, 2), jnp.float32),
        jax.ShapeDtypeStruct((1, 1), jnp.float32),
    ]

    rec1, lsum1 = pl.pallas_call(
        _dec_oh_kernel,
        grid=(GH // SUB,),
        in_specs=[dec_specs[0],
                  pl.BlockSpec((SUB * RT, 1), lambda i: (i, 0)),
                  _const2((C, D))] + dec_specs[2:],
        out_specs=dec_outs,
        out_shape=dec_types,
    )(z_h[1], q2d_h[1], params["codebook"], *dec_args)

    rec0, lsum0 = pl.pallas_call(
        _dec_kernel,
        grid=(GH // SUB,),
        in_specs=dec_specs,
        out_specs=dec_outs,
        out_shape=dec_types,
    )(z_h[0], zq0, *dec_args)

    rec_h = [rec0, rec1]
    lsum_h = [lsum0, lsum1]

    rec = jnp.concatenate(rec_h, axis=0).reshape(BS, J, 2)
    q = jnp.concatenate(q_h, axis=0)
    e_latent_loss = (lsum_h[0][0, 0] + lsum_h[1][0, 0]) / (NZ * D)
    return rec, q, e_latent_loss


# speedup vs baseline: 1.7507x; 1.0941x over previous
import functools
import math

import jax
import jax.numpy as jnp
from jax.experimental import pallas as pl
from jax.experimental.pallas import tpu as pltpu
from jax.experimental.pallas import tpu_sc as plsc

J = 17
T = 34
H = 512
C = 1024
D = 512
BS = 256
BB = 8
G = BS // BB
R = BB * J
RT = BB * T
NZ = BS * T
DH = 32
EPS = 1e-5
SUB = 4

_GW = 16


def _ln(x, g, b):
    m = jnp.mean(x, -1, keepdims=True)
    v = jnp.mean((x - m) ** 2, -1, keepdims=True)
    return (x - m) / jnp.sqrt(v + EPS) * g + b


def _gelu(x):
    return x * 0.5 * (1.0 + jax.lax.erf(x * (1.0 / math.sqrt(2.0))))


def _enc_kernel(coords, w, inv, sw, sb, *rest):
    blocks = [rest[12 * k:12 * (k + 1)] for k in range(4)]
    lng, lnb, mt, mtb, few, feb, cbt = rest[48:55]
    z_ref, q_ref, cbsq_ref = rest[55:58]

    i = pl.program_id(0)

    @pl.when(i == 0)
    def _():
        cbsq_ref[...] = jnp.sum(cbt[...] * cbt[...], axis=0, keepdims=True)

    feats = []
    for hh_ in range(SUB):
        sl = pl.ds(hh_ * R, R)
        wv = w[sl, :]
        feat = jnp.dot(coords[sl, :], sw[...]) + sb[...]
        feats.append(feat * wv + inv[...] * (1.0 - wv))

    for (l1g, l1b, m1, t1b, m2, t2b, l2g, l2b,
         c1w, c1b, c2w, c2b) in blocks:
        ys = [_ln(f, l1g[...], l1b[...]) for f in feats]
        hs = [_gelu(jnp.dot(m1[...], y) + t1b[...]) for y in ys]
        ys = [jnp.dot(m2[...], h) + t2b[...] for h in hs]
        zins = [_ln(f + y, l2g[...], l2b[...]) for f, y in zip(feats, ys)]
        hhs = [_gelu(jnp.dot(zin, c1w[...]) + c1b[...]) for zin in zins]
        zzs = [jnp.dot(hh, c2w[...]) + c2b[...] for hh in hhs]
        feats = [f + y + zz for f, y, zz in zip(feats, ys, zzs)]

    feats = [_ln(f, lng[...], lnb[...]) for f in feats]
    tks = [jnp.dot(mt[...], f) + mtb[...] for f in feats]
    zs = [jnp.dot(tk, few[...]) + feb[...] for tk in tks]

    z = jnp.concatenate(zs, axis=0) if SUB > 1 else zs[0]
    zsq = jnp.sum(z * z, axis=1, keepdims=True)
    d2 = zsq - 2.0 * jnp.dot(z, cbt[...]) + cbsq_ref[...]
    dmin = jnp.min(d2, axis=1, keepdims=True)
    lanes = jax.lax.broadcasted_iota(jnp.int32, d2.shape, 1)
    q = jnp.min(jnp.where(d2 == dmin, lanes, C), axis=1, keepdims=True)

    z_ref[...] = z
    q_ref[...] = q


def _dec_body(zvs, zqvs, md, mdb, dsw, dsb,
              l1g, l1b, dm1, dt1b, dm2, dt2b, l2g, l2b,
              dc1w, dc1b, dc2w, dc2b,
              lng, lnb, rw, rb, rec_ref, lsum_ref):
    i = pl.program_id(0)

    @pl.when(i == 0)
    def _():
        lsum_ref[...] = jnp.zeros_like(lsum_ref)

    acc = None
    for zv, zqv in zip(zvs, zqvs):
        diff = zv - zqv
        p = jnp.sum(diff * diff, axis=(0, 1), keepdims=True)
        acc = p if acc is None else acc + p
    lsum_ref[...] += acc

    decs = [jnp.dot(jnp.dot(md[...], zv + (zqv - zv)) + mdb[...],
                    dsw[...]) + dsb[...]
            for zv, zqv in zip(zvs, zqvs)]

    ys = [_ln(d, l1g[...], l1b[...]) for d in decs]
    hs = [_gelu(jnp.dot(dm1[...], y) + dt1b[...]) for y in ys]
    ys = [jnp.dot(dm2[...], h) + dt2b[...] for h in hs]
    zins = [_ln(d + y, l2g[...], l2b[...]) for d, y in zip(decs, ys)]
    hhs = [_gelu(jnp.dot(zin, dc1w[...]) + dc1b[...]) for zin in zins]
    zzs = [jnp.dot(hh, dc2w[...]) + dc2b[...] for hh in hhs]
    decs = [d + y + zz for d, y, zz in zip(decs, ys, zzs)]

    decs = [_ln(d, lng[...], lnb[...]) for d in decs]
    for c, d in enumerate(decs):
        rec_ref[pl.ds(c * R, R), :] = jnp.dot(d, rw[...]) + rb[...]


def _split(x):
    return [x[c * RT:(c + 1) * RT, :] for c in range(SUB)]


def _dec_kernel(z, zq, *args):
    _dec_body(_split(z[...]), _split(zq[...]), *args)


def _dec_oh_kernel(z, q, cb, *args):
    lanes = jax.lax.broadcasted_iota(jnp.int32, (SUB * RT, C), 1)
    oh = (lanes == q[...]).astype(jnp.float32)
    zqv = jnp.dot(oh, cb[...])
    _dec_body(_split(z[...]), _split(zqv), *args)


def _const2(shape):
    return pl.BlockSpec(shape, lambda i: (0, 0))


_NW = 32
_BPW = NZ // _NW
_CH = 16
_K = 8


def _sc_gather(cb, q, n=NZ):
    mesh = plsc.VectorSubcoreMesh(core_axis_name="c", subcore_axis_name="s")
    grp = _K * _CH
    bpw = n // _NW
    assert bpw * _NW == n and bpw % 8 == 0

    groups = []
    off = 0
    while off < bpw:
        g = []
        while off < bpw and len(g) < _K:
            sz = min(_CH, bpw - off)
            g.append((off, sz))
            off += sz
        groups.append(g)

    @functools.partial(
        pl.kernel,
        out_type=jax.ShapeDtypeStruct((n, D), cb.dtype),
        mesh=mesh,
        scratch_types=[
            pltpu.VMEM((bpw,), jnp.int32),
            pltpu.VMEM((grp, D), jnp.float32),
            pltpu.SemaphoreType.DMA,
        ],
    )
    def kern(cb_hbm, q_hbm, o_hbm, idx_v, rows_v, sem):
        wid = jax.lax.axis_index("s") * 2 + jax.lax.axis_index("c")
        base = wid * bpw
        pltpu.sync_copy(q_hbm.at[pl.ds(base, bpw)], idx_v)
        for g in groups:
            cps = []
            g0 = g[0][0]
            for off, sz in g:
                cps.append(pltpu.async_copy(
                    cb_hbm.at[idx_v.at[pl.ds(off, sz)]],
                    rows_v.at[pl.ds(off - g0, sz)], sem))
            for cp in cps:
                cp.wait()
            gn = g[-1][0] + g[-1][1] - g0
            pltpu.sync_copy(rows_v.at[pl.ds(0, gn)],
                            o_hbm.at[pl.ds(base + g0, gn)])

    return kern(cb, q)


def _row(b):
    return b.reshape(1, -1)


def _bd(wt, bb=BB):
    return jnp.kron(jnp.eye(bb, dtype=wt.dtype), wt.T)


def _colb(b, bb=BB):
    return jnp.tile(b, bb).reshape(-1, 1)


def kernel(joints, joints_feature, cls_logits, params):
    del joints_feature, cls_logits
    coords = joints[:, :, :2].reshape(BS * J, 2)
    w2d = (joints[:, :, 2] != 0).astype(jnp.float32).reshape(BS * J, 1)

    enc_args = [coords, w2d,
                params["invisible_token"].reshape(1, H),
                params["start_embed"]["w"], _row(params["start_embed"]["b"])]
    enc_specs = [
        pl.BlockSpec((SUB * R, 2), lambda i: (i, 0)),
        pl.BlockSpec((SUB * R, 1), lambda i: (i, 0)),
        _const2((1, H)), _const2((2, H)), _const2((1, H)),
    ]
    for p in params["encoder"]:
        enc_args += [
            _row(p["ln1_g"]), _row(p["ln1_b"]),
            _bd(p["tok1"]["w"]), _colb(p["tok1"]["b"]),
            _bd(p["tok2"]["w"]), _colb(p["tok2"]["b"]),
            _row(p["ln2_g"]), _row(p["ln2_b"]),
            p["ch1"]["w"], _row(p["ch1"]["b"]),
            p["ch2"]["w"], _row(p["ch2"]["b"]),
        ]
        enc_specs += [
            _const2((1, H)), _const2((1, H)),
            _const2((BB * 64, R)), _const2((BB * 64, 1)),
            _const2((R, BB * 64)), _const2((R, 1)),
            _const2((1, H)), _const2((1, H)),
            _const2((H, H)), _const2((1, H)),
            _const2((H, H)), _const2((1, H)),
        ]
    enc_args += [
        _row(params["enc_ln_g"]), _row(params["enc_ln_b"]),
        _bd(params["token_mlp"]["w"]), _colb(params["token_mlp"]["b"]),
        params["feature_embed"]["w"], _row(params["feature_embed"]["b"]),
        params["codebook"].T,
    ]
    enc_specs += [
        _const2((1, H)), _const2((1, H)),
        _const2((RT, R)), _const2((RT, 1)),
        _const2((H, D)), _const2((1, D)),
        _const2((D, C)),
    ]

    NH = NZ // 2
    RH = BS * J // 2
    GH = G // 2

    def _enc_half(coords_h, w_h):
        return pl.pallas_call(
            _enc_kernel,
            grid=(GH // SUB,),
            in_specs=enc_specs,
            out_specs=[
                pl.BlockSpec((SUB * RT, D), lambda i: (i, 0)),
                pl.BlockSpec((SUB * RT, 1), lambda i: (i, 0)),
            ],
            out_shape=[
                jax.ShapeDtypeStruct((NH, D), jnp.float32),
                jax.ShapeDtypeStruct((NH, 1), jnp.int32),
            ],
            scratch_shapes=[pltpu.VMEM((1, C), jnp.float32)],
        )(coords_h, w_h, *enc_args[2:])

    z_h, q_h, q2d_h = [], [], []
    for h in range(2):
        zh, qh = _enc_half(coords[h * RH:(h + 1) * RH],
                           w2d[h * RH:(h + 1) * RH])
        z_h.append(zh)
        q2d_h.append(qh)
        q_h.append(qh.reshape(NH))

    zq0 = _sc_gather(params["codebook"], q_h[0], NH)

    dp = params["decoder"][0]
    dec_args = [
        _bd(params["decoder_token_mlp"]["w"]),
        _colb(params["decoder_token_mlp"]["b"]),
        params["decoder_start"]["w"], _row(params["decoder_start"]["b"]),
        _row(dp["ln1_g"]), _row(dp["ln1_b"]),
        _bd(dp["tok1"]["w"]), _colb(dp["tok1"]["b"]),
        _bd(dp["tok2"]["w"]), _colb(dp["tok2"]["b"]),
        _row(dp["ln2_g"]), _row(dp["ln2_b"]),
        dp["ch1"]["w"], _row(dp["ch1"]["b"]),
        dp["ch2"]["w"], _row(dp["ch2"]["b"]),
        _row(params["dec_ln_g"]), _row(params["dec_ln_b"]),
        params["recover_embed"]["w"], _row(params["recover_embed"]["b"]),
    ]
    dec_specs = [
        pl.BlockSpec((SUB * RT, D), lambda i: (i, 0)),
        pl.BlockSpec((SUB * RT, D), lambda i: (i, 0)),
        _const2((R, RT)), _const2((R, 1)),
        _const2((D, DH)), _const2((1, DH)),
        _const2((1, DH)), _const2((1, DH)),
        _const2((BB * 64, R)), _const2((BB * 64, 1)),
        _const2((R, BB * 64)), _const2((R, 1)),
        _const2((1, DH)), _const2((1, DH)),
        _const2((DH, 64)), _const2((1, 64)),
        _const2((64, DH)), _const2((1, DH)),
        _const2((1, DH)), _const2((1, DH)),
        _const2((DH, 2)), _const2((1, 2)),
    ]

    dec_outs = [
        pl.BlockSpec((SUB * R, 2), lambda i: (i, 0)),
        pl.BlockSpec((1, 1), lambda i: (0, 0)),
    ]
    dec_types = [
        jax.ShapeDtypeStruct((RH, 2), jnp.float32),
        jax.ShapeDtypeStruct((1, 1), jnp.float32),
    ]

    rec1, lsum1 = pl.pallas_call(
        _dec_oh_kernel,
        grid=(GH // SUB,),
        in_specs=[dec_specs[0],
                  pl.BlockSpec((SUB * RT, 1), lambda i: (i, 0)),
                  _const2((C, D))] + dec_specs[2:],
        out_specs=dec_outs,
        out_shape=dec_types,
    )(z_h[1], q2d_h[1], params["codebook"], *dec_args)

    rec0, lsum0 = pl.pallas_call(
        _dec_kernel,
        grid=(GH // SUB,),
        in_specs=dec_specs,
        out_specs=dec_outs,
        out_shape=dec_types,
    )(z_h[0], zq0, *dec_args)

    rec_h = [rec0, rec1]
    lsum_h = [lsum0, lsum1]

    rec = jnp.concatenate(rec_h, axis=0).reshape(BS, J, 2)
    q = jnp.concatenate(q_h, axis=0)
    e_latent_loss = (lsum_h[0][0, 0] + lsum_h[1][0, 0]) / (NZ * D)
    return rec, q, e_latent_loss
